# baseline (speedup 1.0000x reference)
import functools

import jax
import jax.numpy as jnp
from jax import lax
from jax.experimental import pallas as pl
from jax.experimental.pallas import tpu as pltpu
from jax.experimental.pallas import tpu_sc as plsc

_B = 16384
_H = _B // 2
_EMB = 64
_N_LABELS = 64
_TC_BLK = 4096


@functools.cache
def _make_gather2(h):
    info = plsc.get_sparse_core_info()
    nw = info.num_cores * info.num_subcores
    b_per_w = _H // nw
    mesh = plsc.VectorSubcoreMesh(core_axis_name="c", subcore_axis_name="s")

    @functools.partial(
        pl.kernel,
        mesh=mesh,
        out_type=[
            jax.ShapeDtypeStruct((_H, _EMB), jnp.float32),
            jax.ShapeDtypeStruct((_H, _EMB), jnp.float32),
        ],
        scratch_types=[
            pltpu.VMEM((b_per_w,), jnp.int32),
            pltpu.VMEM((b_per_w,), jnp.int32),
            pltpu.VMEM((b_per_w, _EMB), jnp.float32),
            pltpu.VMEM((b_per_w, _EMB), jnp.float32),
            pltpu.SemaphoreType.DMA,
        ],
    )
    def gather2(cells_hbm, genes_hbm, cell_tab, gene_tab,
                cell_out, gene_out,
                cidx_v, gidx_v, cstage, gstage, sem):
        wid = lax.axis_index("s") * info.num_cores + lax.axis_index("c")
        base = wid * b_per_w
        src = h * _H + base

        pltpu.sync_copy(cells_hbm.at[pl.ds(src, b_per_w)], cidx_v)
        pltpu.sync_copy(genes_hbm.at[pl.ds(src, b_per_w)], gidx_v)

        def fetch(g, _):
            cv = cidx_v[pl.ds(g * 16, 16)]
            gv = gidx_v[pl.ds(g * 16, 16)]
            ct = lax.shift_right_logical(cv, 3)
            cs = jnp.bitwise_and(cv, 7)
            gt = lax.shift_right_logical(gv, 3)
            gs = jnp.bitwise_and(gv, 7)
            for k in range(16):
                pltpu.async_copy(
                    cell_tab.at[ct[k], cs[k]],
                    cstage.at[g * 16 + k], sem)
                pltpu.async_copy(
                    gene_tab.at[gt[k], gs[k]],
                    gstage.at[g * 16 + k], sem)
            return 0

        lax.fori_loop(0, b_per_w // 16, fetch, 0)
        pltpu.make_async_copy(
            cell_out.at[pl.ds(base, b_per_w)], cstage, sem).wait()
        pltpu.make_async_copy(
            gene_out.at[pl.ds(base, b_per_w)], gstage, sem).wait()
        pltpu.sync_copy(cstage, cell_out.at[pl.ds(base, b_per_w)])
        pltpu.sync_copy(gstage, gene_out.at[pl.ds(base, b_per_w)])

    return gather2


def _tc_body(ce_ref, ge_ref, w_ref, eye_ref, qz_ref, pz_ref, ce_t_ref,
             rec_ref):
    ce = ce_ref[...]
    ge = ge_ref[...]
    w = w_ref[...]
    eye = eye_ref[...]
    pz_logit = lax.dot_general(w, ce, (((1,), (1,)), ((), ())),
                               preferred_element_type=jnp.float32)
    qz_logit = lax.dot_general(w, ce * ge, (((1,), (1,)), ((), ())),
                               preferred_element_type=jnp.float32)

    lab = lax.broadcasted_iota(jnp.int32, qz_logit.shape, 0)
    col_max = jnp.max(qz_logit, axis=0, keepdims=True)
    amax = jnp.min(jnp.where(qz_logit == col_max, lab, _N_LABELS),
                   axis=0, keepdims=True)
    onehot = (lab == amax).astype(jnp.float32)
    rec_ref[...] = lax.dot_general(w, onehot, (((0,), (0,)), ((), ())),
                                   preferred_element_type=jnp.float32)

    qe = jnp.exp(qz_logit - col_max)
    qz_ref[...] = qe / jnp.sum(qe, axis=0, keepdims=True)
    pe = jnp.exp(pz_logit - jnp.max(pz_logit, axis=0, keepdims=True))
    pz_ref[...] = pe / jnp.sum(pe, axis=0, keepdims=True)
    ce_t_ref[...] = lax.dot_general(eye, ce, (((1,), (1,)), ((), ())),
                                    preferred_element_type=jnp.float32)


def _tc_body_aliased(ce_ref, ge_ref, w_ref, eye_ref, a0, a1, a2, a3,
                     qz_ref, pz_ref, ce_t_ref, rec_ref):
    del a0, a1, a2, a3
    _tc_body(ce_ref, ge_ref, w_ref, eye_ref, qz_ref, pz_ref, ce_t_ref,
             rec_ref)


def _tc_tail_half(ce_h, ge_h, w_ct, h, prev=None):
    n_blk = _H // _TC_BLK
    blk = pl.BlockSpec((_TC_BLK, _EMB), lambda i: (i, 0))
    wspec = pl.BlockSpec((_N_LABELS, _EMB), lambda i: (0, 0))
    out_l = jax.ShapeDtypeStruct((_N_LABELS, _B), jnp.float32)
    out_e = jax.ShapeDtypeStruct((_EMB, _B), jnp.float32)
    eye = jnp.eye(_EMB, dtype=jnp.float32)
    off = h * n_blk
    out_specs = (
        [pl.BlockSpec((_N_LABELS, _TC_BLK), lambda i, off=off: (0, off + i))]
        * 2
        + [pl.BlockSpec((_EMB, _TC_BLK), lambda i, off=off: (0, off + i))]
        * 2)
    if prev is None:
        return pl.pallas_call(
            _tc_body,
            grid=(n_blk,),
            in_specs=[blk, blk, wspec,
                      pl.BlockSpec((_EMB, _EMB), lambda i: (0, 0))],
            out_specs=out_specs,
            out_shape=[out_l, out_l, out_e, out_e],
        )(ce_h, ge_h, w_ct, eye)
    anyspec = pl.BlockSpec(memory_space=pltpu.MemorySpace.HBM)
    return pl.pallas_call(
        _tc_body_aliased,
        grid=(n_blk,),
        in_specs=[blk, blk, wspec,
                  pl.BlockSpec((_EMB, _EMB), lambda i: (0, 0))]
        + [anyspec] * 4,
        out_specs=out_specs,
        out_shape=[out_l, out_l, out_e, out_e],
        input_output_aliases={4: 0, 5: 1, 6: 2, 7: 3},
    )(ce_h, ge_h, w_ct, eye, *prev)


def kernel(cells, genes, w_cell_table, w_gene_table, W_ct):
    ct3 = w_cell_table.reshape(-1, 8, _EMB)
    gt3 = w_gene_table.reshape(-1, 8, _EMB)
    ce1, ge1 = _make_gather2(0)(cells, genes, ct3, gt3)
    ce2, ge2 = _make_gather2(1)(cells, genes, ct3, gt3)
    part = _tc_tail_half(ce1, ge1, W_ct, 0)
    qz_t, pz_t, ce_t, rec_t = _tc_tail_half(ce2, ge2, W_ct, 1, prev=part)
    return (qz_t.T, pz_t.T, ce_t.T, rec_t.T)


# speedup vs baseline: 1.0013x; 1.0013x over previous
import functools

import jax
import jax.numpy as jnp
from jax import lax
from jax.experimental import pallas as pl
from jax.experimental.pallas import tpu as pltpu
from jax.experimental.pallas import tpu_sc as plsc

_B = 16384
_EMB = 64
_N_LABELS = 64
_TC_BLK = 4096


@functools.cache
def _make_gather2():
    info = plsc.get_sparse_core_info()
    nw = info.num_cores * info.num_subcores
    b_per_w = _B // nw
    mesh = plsc.VectorSubcoreMesh(core_axis_name="c", subcore_axis_name="s")

    @functools.partial(
        pl.kernel,
        mesh=mesh,
        out_type=[
            jax.ShapeDtypeStruct((_B, _EMB), jnp.float32),
            jax.ShapeDtypeStruct((_B, _EMB), jnp.float32),
        ],
        scratch_types=[
            pltpu.VMEM((b_per_w,), jnp.int32),
            pltpu.VMEM((b_per_w,), jnp.int32),
            pltpu.VMEM((b_per_w // 2, _EMB), jnp.float32),
            pltpu.VMEM((b_per_w // 2, _EMB), jnp.float32),
            pltpu.SemaphoreType.DMA,
        ],
    )
    def gather2(cells_hbm, genes_hbm, cell_tab, gene_tab,
                cell_out, gene_out,
                cidx_v, gidx_v, cstage, gstage, sem):
        wid = lax.axis_index("s") * info.num_cores + lax.axis_index("c")
        base = wid * b_per_w

        pltpu.sync_copy(cells_hbm.at[pl.ds(base, b_per_w)], cidx_v)
        pltpu.sync_copy(genes_hbm.at[pl.ds(base, b_per_w)], gidx_v)
        half = b_per_w // 2

        for p in range(2):
            def fetch(g, _, p=p):
                off = p * half
                cv = cidx_v[pl.ds(off + g * 16, 16)]
                gv = gidx_v[pl.ds(off + g * 16, 16)]
                ct = lax.shift_right_logical(cv, 3)
                cs = jnp.bitwise_and(cv, 7)
                gt = lax.shift_right_logical(gv, 3)
                gs = jnp.bitwise_and(gv, 7)
                for k in range(16):
                    pltpu.async_copy(
                        cell_tab.at[ct[k], cs[k]],
                        cstage.at[g * 16 + k], sem)
                    pltpu.async_copy(
                        gene_tab.at[gt[k], gs[k]],
                        gstage.at[g * 16 + k], sem)
                return 0

            lax.fori_loop(0, half // 16, fetch, 0)
            pltpu.make_async_copy(
                cell_out.at[pl.ds(base + p * half, half)], cstage, sem).wait()
            pltpu.make_async_copy(
                gene_out.at[pl.ds(base + p * half, half)], gstage, sem).wait()
            pltpu.sync_copy(cstage, cell_out.at[pl.ds(base + p * half, half)])
            pltpu.sync_copy(gstage, gene_out.at[pl.ds(base + p * half, half)])

    return gather2


def _tc_body(ce_ref, ge_ref, w_ref, eye_ref, qz_ref, pz_ref, ce_t_ref,
             rec_ref):
    ce = ce_ref[...]
    ge = ge_ref[...]
    w = w_ref[...]
    eye = eye_ref[...]
    pz_logit = lax.dot_general(w, ce, (((1,), (1,)), ((), ())),
                               preferred_element_type=jnp.float32)
    qz_logit = lax.dot_general(w, ce * ge, (((1,), (1,)), ((), ())),
                               preferred_element_type=jnp.float32)

    lab = lax.broadcasted_iota(jnp.int32, qz_logit.shape, 0)
    col_max = jnp.max(qz_logit, axis=0, keepdims=True)
    amax = jnp.min(jnp.where(qz_logit == col_max, lab, _N_LABELS),
                   axis=0, keepdims=True)
    onehot = (lab == amax).astype(jnp.float32)
    rec_ref[...] = lax.dot_general(w, onehot, (((0,), (0,)), ((), ())),
                                   preferred_element_type=jnp.float32)

    qe = jnp.exp(qz_logit - col_max)
    qz_ref[...] = qe / jnp.sum(qe, axis=0, keepdims=True)
    pe = jnp.exp(pz_logit - jnp.max(pz_logit, axis=0, keepdims=True))
    pz_ref[...] = pe / jnp.sum(pe, axis=0, keepdims=True)
    ce_t_ref[...] = lax.dot_general(eye, ce, (((1,), (1,)), ((), ())),
                                    preferred_element_type=jnp.float32)


def _tc_tail(ce, ge, w_ct):
    n_blk = _B // _TC_BLK
    blk = pl.BlockSpec((_TC_BLK, _EMB), lambda i: (i, 0))
    wspec = pl.BlockSpec((_N_LABELS, _EMB), lambda i: (0, 0))
    out_l = jax.ShapeDtypeStruct((_N_LABELS, _B), jnp.float32)
    out_e = jax.ShapeDtypeStruct((_EMB, _B), jnp.float32)
    eye = jnp.eye(_EMB, dtype=jnp.float32)
    return pl.pallas_call(
        _tc_body,
        grid=(n_blk,),
        in_specs=[blk, blk, wspec,
                  pl.BlockSpec((_EMB, _EMB), lambda i: (0, 0))],
        out_specs=[pl.BlockSpec((_N_LABELS, _TC_BLK), lambda i: (0, i))] * 2
        + [pl.BlockSpec((_EMB, _TC_BLK), lambda i: (0, i))] * 2,
        out_shape=[out_l, out_l, out_e, out_e],
    )(ce, ge, w_ct, eye)


def kernel(cells, genes, w_cell_table, w_gene_table, W_ct):
    ct3 = w_cell_table.reshape(-1, 8, _EMB)
    gt3 = w_gene_table.reshape(-1, 8, _EMB)
    ce, ge = _make_gather2()(cells, genes, ct3, gt3)
    qz_t, pz_t, ce_t, rec_t = _tc_tail(ce, ge, W_ct)
    return (qz_t.T, pz_t.T, ce_t.T, rec_t.T)
